# baseline (speedup 1.0000x reference)
import functools

import jax
import jax.numpy as jnp
from jax.experimental import pallas as pl
from jax.experimental.pallas import tpu as pltpu

N = 100000
NODE_IN = 128
D_E = 16
HID = 256
NODE_OUT = 128
ROW_BLK = 1000


def _mlp_body(x_ref, a1_ref, a2_ref, a3_ref, w1x_ref, w1a_ref, w1b_ref,
              w1c_ref, b1_ref, w2_ref, b2_ref, out_ref):
    h = jnp.dot(x_ref[...], w1x_ref[...], preferred_element_type=jnp.float32)
    h += jnp.dot(a1_ref[...], w1a_ref[...], preferred_element_type=jnp.float32)
    h += jnp.dot(a2_ref[...], w1b_ref[...], preferred_element_type=jnp.float32)
    h += jnp.dot(a3_ref[...], w1c_ref[...], preferred_element_type=jnp.float32)
    h = jnp.maximum(h + b1_ref[...], 0.0)
    out_ref[...] = jnp.dot(h, w2_ref[...], preferred_element_type=jnp.float32) + b2_ref[...]


@jax.jit
def _mlp(x, a1, a2, a3, W1, b1, W2, b2):
    w1x = W1[:NODE_IN]
    w1a = W1[NODE_IN:NODE_IN + D_E]
    w1b = W1[NODE_IN + D_E:NODE_IN + 2 * D_E]
    w1c = W1[NODE_IN + 2 * D_E:]
    grid = (N // ROW_BLK,)
    row_spec = lambda d: pl.BlockSpec((ROW_BLK, d), lambda i: (i, 0))
    full = lambda a, b: pl.BlockSpec((a, b), lambda i: (0, 0))
    return pl.pallas_call(
        _mlp_body,
        grid=grid,
        in_specs=[
            row_spec(NODE_IN), row_spec(D_E), row_spec(D_E), row_spec(D_E),
            full(NODE_IN, HID), full(D_E, HID), full(D_E, HID), full(D_E, HID),
            full(1, HID), full(HID, NODE_OUT), full(1, NODE_OUT),
        ],
        out_specs=row_spec(NODE_OUT),
        out_shape=jax.ShapeDtypeStruct((N, NODE_OUT), jnp.float32),
    )(x, a1, a2, a3, w1x, w1a, w1b, w1c, b1.reshape(1, HID), W2,
      b2.reshape(1, NODE_OUT))


def kernel(x, edge_index, edge_attr, u, batch, W1, b1, W2, b2):
    col = edge_index[1]
    n = x.shape[0]
    out1 = jnp.zeros((n, D_E), dtype=x.dtype).at[col].add(edge_attr)
    out2 = jnp.zeros((n, D_E), dtype=x.dtype).at[col].max(edge_attr)
    cnt = jnp.zeros((n,), dtype=x.dtype).at[col].add(1.0)
    out3 = out1 / jnp.clip(cnt, 1.0, None)[:, None]
    return _mlp(x, out1, out2, out3, W1, b1, W2, b2)


# speedup vs baseline: 1.7430x; 1.7430x over previous
import functools

import jax
import jax.numpy as jnp
from jax import lax
from jax.experimental import pallas as pl
from jax.experimental.pallas import tpu as pltpu
from jax.experimental.pallas import tpu_sc as plsc

N = 100000
E = 3200000
NODE_IN = 128
D_E = 16
HID = 256
NODE_OUT = 128

NC = 2
NS = 16

HALF = N // NC
LPAD = 50048
TRASH = 1024
LTAB = LPAD + TRASH
OUT_ROWS = 100048

WB = LPAD // NS
ZCH = LTAB // NS

SUB = 128
NSUB = 16
CHUNK = SUB * NSUB
NFULL = E // CHUNK
TAILE = E - NFULL * CHUNK
TAILG = TAILE // SUB
CPT = NFULL // NS + 1

ROW_BLK = 1000


def _filter_group(idx2, j, ngroups, lo, tid, trash_rot):
    iota = lax.iota(jnp.int32, 16)
    for i in range(8):
        v = idx2[j, pl.ds(i * 16, 16)]
        lv = v - lo
        m = jnp.logical_and(lv >= 0, lv < HALF)
        tv = LPAD + jnp.bitwise_and(j * SUB + i * 16 + trash_rot + iota,
                                    TRASH - 1)
        idx2[j, pl.ds(i * 16, 16)] = jnp.where(m, lv, tv)


def _sc_body(col4, attr, out1, cnt1, idx2, attr_v, ones_v, zrow_v, zcnt_v,
             acc, cnt):
    cid = lax.axis_index("c")
    tid = lax.axis_index("s")

    if True:
        def zr(i, _):
            zrow_v[i, :] = jnp.zeros((16,), jnp.float32)
            return _
        lax.fori_loop(0, zrow_v.shape[0], zr, None)

        def zc(i, _):
            zcnt_v[pl.ds(i * 16, 16)] = jnp.zeros((16,), jnp.float32)
            return _
        lax.fori_loop(0, zcnt_v.shape[0] // 16, zc, None)

        def oi(i, _):
            ones_v[pl.ds(i * 16, 16)] = jnp.ones((16,), jnp.float32)
            return _
        lax.fori_loop(0, 8, oi, None)

        for k in range(3):
            pltpu.sync_copy(zrow_v,
                            acc.at[pl.ds(tid * ZCH + k * 1064, 1064)])
        pltpu.sync_copy(zcnt_v.at[pl.ds(0, ZCH)], cnt.at[pl.ds(tid * ZCH, ZCH)])
        plsc.subcore_barrier()

        lo = cid * HALF
        trash_rot = tid * 64

        def do_chunk(g, ngroups):
            pltpu.sync_copy(col4.at[pl.ds(g * NSUB, ngroups)],
                            idx2.at[pl.ds(0, ngroups)])
            pltpu.sync_copy(attr.at[pl.ds(g * CHUNK, ngroups * SUB)],
                            attr_v.at[pl.ds(0, ngroups * SUB)])
            for j in range(ngroups):
                _filter_group(idx2, j, ngroups, lo, tid, trash_rot)
            for j in range(ngroups):
                pltpu.sync_copy(attr_v.at[pl.ds(j * SUB, SUB)],
                                acc.at[idx2.at[j]], add=True)
                pltpu.sync_copy(ones_v, cnt.at[idx2.at[j]], add=True)

        nchunks = jnp.where(tid == NS - 1, CPT - 6, CPT)

        def chunk_body(k, _):
            do_chunk(tid * CPT + k, NSUB)
            return _
        lax.fori_loop(0, nchunks, chunk_body, None)

        @pl.when(tid == NS - 1)
        def _tail():
            do_chunk(NFULL, TAILG)

        plsc.subcore_barrier()

        last0 = jnp.logical_and(cid == 0, tid == NS - 1)
        gbase = cid * HALF + tid * WB

        @pl.when(last0)
        def _wb_short():
            nrows = WB - (LPAD - HALF)
            pltpu.sync_copy(acc.at[pl.ds(tid * WB, nrows)],
                            out1.at[pl.ds(gbase, nrows)])
            pltpu.sync_copy(cnt.at[pl.ds(tid * WB, nrows)],
                            zcnt_v.at[pl.ds(0, nrows)])
            pltpu.sync_copy(zcnt_v.at[pl.ds(0, nrows)],
                            cnt1.at[pl.ds(gbase, nrows)])

        @pl.when(jnp.logical_not(last0))
        def _wb_full():
            pltpu.sync_copy(acc.at[pl.ds(tid * WB, WB)],
                            out1.at[pl.ds(gbase, WB)])
            pltpu.sync_copy(cnt.at[pl.ds(tid * WB, WB)],
                            zcnt_v.at[pl.ds(0, WB)])
            pltpu.sync_copy(zcnt_v.at[pl.ds(0, WB)],
                            cnt1.at[pl.ds(gbase, WB)])

@jax.jit
def _sc_scatter(col4, edge_attr):
    return pl.kernel(
        _sc_body,
        out_type=[
            jax.ShapeDtypeStruct((OUT_ROWS, D_E), jnp.float32),
            jax.ShapeDtypeStruct((OUT_ROWS,), jnp.float32),
        ],
        mesh=plsc.VectorSubcoreMesh(core_axis_name="c", subcore_axis_name="s"),
        compiler_params=pltpu.CompilerParams(use_tc_tiling_on_sc=False),
        scratch_types=[
            pltpu.VMEM((NSUB, SUB), jnp.int32),
            pltpu.VMEM((CHUNK, D_E), jnp.float32),
            pltpu.VMEM((SUB,), jnp.float32),
            pltpu.VMEM((1064, D_E), jnp.float32),
            pltpu.VMEM((3200,), jnp.float32),
            pltpu.VMEM_SHARED((LTAB, D_E), jnp.float32),
            pltpu.VMEM_SHARED((LTAB,), jnp.float32),
        ],
    )(col4, edge_attr)


def _mlp_body(x_ref, a1_ref, a2_ref, cnt_ref, w1x_ref, w1a_ref, w1b_ref,
              w1c_ref, b1_ref, w2_ref, b2_ref, out_ref):
    a1 = a1_ref[...]
    cnt = cnt_ref[...][:, 0]
    a3 = a1 / jnp.clip(cnt, 1.0, None)[:, None]
    h = jnp.dot(x_ref[...], w1x_ref[...], preferred_element_type=jnp.float32)
    h += jnp.dot(a1, w1a_ref[...], preferred_element_type=jnp.float32)
    h += jnp.dot(a2_ref[...], w1b_ref[...], preferred_element_type=jnp.float32)
    h += jnp.dot(a3, w1c_ref[...], preferred_element_type=jnp.float32)
    h = jnp.maximum(h + b1_ref[...], 0.0)
    out_ref[...] = jnp.dot(h, w2_ref[...], preferred_element_type=jnp.float32) + b2_ref[...]


@jax.jit
def _mlp(x, a1, a2, cnt, W1, b1, W2, b2):
    w1x = W1[:NODE_IN]
    w1a = W1[NODE_IN:NODE_IN + D_E]
    w1b = W1[NODE_IN + D_E:NODE_IN + 2 * D_E]
    w1c = W1[NODE_IN + 2 * D_E:]
    grid = (N // ROW_BLK,)
    row = lambda d: pl.BlockSpec((ROW_BLK, d), lambda i: (i, 0))
    full = lambda a, b: pl.BlockSpec((a, b), lambda i: (0, 0))
    return pl.pallas_call(
        _mlp_body,
        grid=grid,
        in_specs=[
            row(NODE_IN), row(D_E), row(D_E), row(1),
            full(NODE_IN, HID), full(D_E, HID), full(D_E, HID), full(D_E, HID),
            full(1, HID), full(HID, NODE_OUT), full(1, NODE_OUT),
        ],
        out_specs=row(NODE_OUT),
        out_shape=jax.ShapeDtypeStruct((N, NODE_OUT), jnp.float32),
    )(x, a1, a2, cnt, w1x, w1a, w1b, w1c, b1.reshape(1, HID), W2,
      b2.reshape(1, NODE_OUT))


def kernel(x, edge_index, edge_attr, u, batch, W1, b1, W2, b2):
    col = edge_index[1]
    col4 = col.reshape(E // SUB, SUB)
    a1, cnt1 = _sc_scatter(col4, edge_attr)
    out2 = jnp.zeros((N, D_E), dtype=x.dtype).at[col].max(edge_attr)
    return _mlp(x, a1, out2, cnt1.reshape(OUT_ROWS, 1), W1, b1, W2, b2)


# speedup vs baseline: 2.3964x; 1.3749x over previous
import functools

import jax
import jax.numpy as jnp
from jax import lax
from jax.experimental import pallas as pl
from jax.experimental.pallas import tpu as pltpu
from jax.experimental.pallas import tpu_sc as plsc

N = 100000
E = 3200000
NODE_IN = 128
D_E = 16
HID = 256
NODE_OUT = 128

NC = 2
NS = 16
HALF = N // NC
TR = 3128
TPAD = 3136
OUT_ROWS = 100048

CHUNK = 2000
NPAIR = (E // CHUNK) // 2
GROUPS = CHUNK // 16

RING = 64

ROW_BLK = 1000


def _sc_body(col_h, attr_h, out1, out2, cnt1, cbufA, cbufB, lvb, eidb, gbig,
             macc, sacc, cntacc, csemA, csemB):
    cid = lax.axis_index("c")
    tid = lax.axis_index("s")
    base = cid * HALF + tid * TR
    iota = lax.iota(jnp.int32, 16)
    zeros16 = jnp.zeros((16,), jnp.float32)
    ones16 = jnp.ones((16,), jnp.float32)

    def zr(r, c):
        macc[r, :] = zeros16
        sacc[r, :] = zeros16
        return c
    lax.fori_loop(0, TPAD, zr, 0)

    def zc(r, c):
        cntacc[pl.ds(r * 16, 16)] = zeros16
        return c
    lax.fori_loop(0, TPAD // 16, zc, 0)

    def scan_group(cbuf, i, ebase, moff):
        v = cbuf[pl.ds(i * 16, 16)]
        lv = v - base
        m = plsc.bitcast(lv, jnp.uint32) < jnp.uint32(TR)
        mi = m.astype(jnp.int32)
        ranks = plsc.cumsum(mi) - 1
        addr = moff + ranks
        row = jnp.bitwise_and(lax.shift_right_logical(addr, 7), RING - 1)
        lane = jnp.bitwise_and(addr, 127)
        plsc.store_scatter(lvb, [row, lane], lv, mask=m)
        plsc.store_scatter(eidb, [row, lane], ebase + i * 16 + iota, mask=m)
        return moff + jnp.sum(mi)

    def process_group(lrow, i, goff):
        lv = lvb[lrow, pl.ds(i * 16, 16)]
        cnts, _ = plsc.scan_count(lv)
        maxc = jnp.max(cnts)
        gidx = goff + i * 16 + iota

        def round_body(r2, c):
            rm = cnts == r2
            plsc.addupdate_scatter(cntacc, [lv], ones16, mask=rm)
            for d in range(D_E):
                dv = jnp.full((16,), d, jnp.int32)
                val = plsc.load_gather(gbig, [gidx, dv], mask=rm)
                cur = plsc.load_gather(macc, [lv, dv], mask=rm)
                plsc.store_scatter(macc, [lv, dv], jnp.maximum(cur, val),
                                   mask=rm)
                plsc.addupdate_scatter(sacc, [lv, dv], val, mask=rm)
            return c
        lax.fori_loop(0, maxc + 1, round_body, 0)

    def drain_full(moff, dmoff):
        nfull = lax.shift_right_logical(moff - dmoff, 7)

        def body(q, dmo):
            row = jnp.bitwise_and(lax.shift_right_logical(dmo, 7), RING - 1)
            pltpu.sync_copy(attr_h.at[eidb.at[row]], gbig)
            for i in range(8):
                process_group(row, i, 0)
            return dmo + 128
        return lax.fori_loop(0, nfull, body, dmoff)

    def scan_chunk(cbuf, cstart, offs):
        moff, dmoff = offs
        dmoff = drain_full(moff, dmoff)

        def g(i, mo):
            return scan_group(cbuf, i, cstart, mo)
        moff = lax.fori_loop(0, GROUPS, g, moff)
        return moff, dmoff

    pltpu.async_copy(col_h.at[pl.ds(0, CHUNK)], cbufA, csemA)

    def pair_body(k, offs):
        c0 = 2 * k
        pltpu.async_copy(col_h.at[pl.ds((c0 + 1) * CHUNK, CHUNK)], cbufB,
                         csemB)
        pltpu.make_async_copy(col_h.at[pl.ds(0, CHUNK)], cbufA, csemA).wait()
        offs = scan_chunk(cbufA, c0 * CHUNK, offs)

        @pl.when(k < NPAIR - 1)
        def _():
            pltpu.async_copy(col_h.at[pl.ds((c0 + 2) * CHUNK, CHUNK)], cbufA,
                             csemA)
        pltpu.make_async_copy(col_h.at[pl.ds(0, CHUNK)], cbufB, csemB).wait()
        offs = scan_chunk(cbufB, (c0 + 1) * CHUNK, offs)
        return offs

    moff, dmoff = lax.fori_loop(0, NPAIR, pair_body,
                                (jnp.int32(0), jnp.int32(0)))

    mtarget = jnp.bitwise_and(moff + 127, ~jnp.int32(127))
    for j in range(8):
        addr = moff + j * 16 + iota
        pm = addr < mtarget
        row = jnp.bitwise_and(lax.shift_right_logical(addr, 7), RING - 1)
        lane = jnp.bitwise_and(addr, 127)
        padlv = TR + jnp.bitwise_and(iota, 7)
        plsc.store_scatter(lvb, [row, lane], padlv, mask=pm)
        plsc.store_scatter(eidb, [row, lane], jnp.zeros((16,), jnp.int32),
                           mask=pm)
    moff = mtarget

    dmoff = drain_full(moff, dmoff)

    gbase = cid * HALF + tid * TR
    last0 = jnp.logical_and(cid == 0, tid == NS - 1)

    @pl.when(last0)
    def _wb_short():
        nr = TR - (NS * TR - HALF)
        pltpu.sync_copy(sacc.at[pl.ds(0, nr)], out1.at[pl.ds(gbase, nr)])
        pltpu.sync_copy(macc.at[pl.ds(0, nr)], out2.at[pl.ds(gbase, nr)])
        pltpu.sync_copy(cntacc.at[pl.ds(0, nr)], cnt1.at[pl.ds(gbase, nr)])

    @pl.when(jnp.logical_not(last0))
    def _wb_full():
        pltpu.sync_copy(sacc.at[pl.ds(0, TR)], out1.at[pl.ds(gbase, TR)])
        pltpu.sync_copy(macc.at[pl.ds(0, TR)], out2.at[pl.ds(gbase, TR)])
        pltpu.sync_copy(cntacc.at[pl.ds(0, TR)], cnt1.at[pl.ds(gbase, TR)])


@jax.jit
def _sc_scatter(col, edge_attr):
    return pl.kernel(
        _sc_body,
        out_type=[
            jax.ShapeDtypeStruct((OUT_ROWS, D_E), jnp.float32),
            jax.ShapeDtypeStruct((OUT_ROWS, D_E), jnp.float32),
            jax.ShapeDtypeStruct((OUT_ROWS,), jnp.float32),
        ],
        mesh=plsc.VectorSubcoreMesh(core_axis_name="c", subcore_axis_name="s"),
        compiler_params=pltpu.CompilerParams(use_tc_tiling_on_sc=False,
                                             needs_layout_passes=False),
        scratch_types=[
            pltpu.VMEM((CHUNK,), jnp.int32),
            pltpu.VMEM((CHUNK,), jnp.int32),
            pltpu.VMEM((RING, 128), jnp.int32),
            pltpu.VMEM((RING, 128), jnp.int32),
            pltpu.VMEM((128, D_E), jnp.float32),
            pltpu.VMEM((TPAD, D_E), jnp.float32),
            pltpu.VMEM((TPAD, D_E), jnp.float32),
            pltpu.VMEM((TPAD,), jnp.float32),
            pltpu.SemaphoreType.DMA,
            pltpu.SemaphoreType.DMA,
        ],
    )(col, edge_attr)


def _mlp_body(x_ref, a1_ref, a2_ref, cnt_ref, w1x_ref, w1a_ref, w1b_ref,
              w1c_ref, b1_ref, w2_ref, b2_ref, out_ref):
    a1 = a1_ref[...]
    cnt = cnt_ref[...][:, 0]
    a3 = a1 / jnp.clip(cnt, 1.0, None)[:, None]
    h = jnp.dot(x_ref[...], w1x_ref[...], preferred_element_type=jnp.float32)
    h += jnp.dot(a1, w1a_ref[...], preferred_element_type=jnp.float32)
    h += jnp.dot(a2_ref[...], w1b_ref[...], preferred_element_type=jnp.float32)
    h += jnp.dot(a3, w1c_ref[...], preferred_element_type=jnp.float32)
    h = jnp.maximum(h + b1_ref[...], 0.0)
    out_ref[...] = jnp.dot(h, w2_ref[...], preferred_element_type=jnp.float32) + b2_ref[...]


@jax.jit
def _mlp(x, a1, a2, cnt, W1, b1, W2, b2):
    w1x = W1[:NODE_IN]
    w1a = W1[NODE_IN:NODE_IN + D_E]
    w1b = W1[NODE_IN + D_E:NODE_IN + 2 * D_E]
    w1c = W1[NODE_IN + 2 * D_E:]
    grid = (N // ROW_BLK,)
    row = lambda d: pl.BlockSpec((ROW_BLK, d), lambda i: (i, 0))
    full = lambda a, b: pl.BlockSpec((a, b), lambda i: (0, 0))
    return pl.pallas_call(
        _mlp_body,
        grid=grid,
        in_specs=[
            row(NODE_IN), row(D_E), row(D_E), row(1),
            full(NODE_IN, HID), full(D_E, HID), full(D_E, HID), full(D_E, HID),
            full(1, HID), full(HID, NODE_OUT), full(1, NODE_OUT),
        ],
        out_specs=row(NODE_OUT),
        out_shape=jax.ShapeDtypeStruct((N, NODE_OUT), jnp.float32),
    )(x, a1, a2, cnt, w1x, w1a, w1b, w1c, b1.reshape(1, HID), W2,
      b2.reshape(1, NODE_OUT))


def kernel(x, edge_index, edge_attr, u, batch, W1, b1, W2, b2):
    col = edge_index[1]
    a1, a2, cnt1 = _sc_scatter(col, edge_attr)
    return _mlp(x, a1, a2, cnt1.reshape(OUT_ROWS, 1), W1, b1, W2, b2)
